# baseline (speedup 1.0000x reference)
import functools

import jax
import jax.numpy as jnp
from jax import lax
from jax.experimental import pallas as pl
from jax.experimental.pallas import tpu as pltpu
from jax.experimental.pallas import tpu_sc as plsc

N = 10000
E = 160000
G = 128
EPAD = 163840
NBLK = EPAD // G
NACC = 10112
ZROWS = NACC // 16
R = 2000


def _sc_scatter(nc, C, const_ones=False):
    n_out = 2 * nc
    BT = NBLK // 32
    HB = BT
    NH = BT // HB
    U = 8
    mesh = plsc.VectorSubcoreMesh(core_axis_name="c", subcore_axis_name="s",
                                  num_cores=2, num_subcores=16)

    @functools.partial(
        pl.kernel,
        out_type=jax.ShapeDtypeStruct((n_out, NACC, C), jnp.float32),
        mesh=mesh,
        scratch_types=[
            pltpu.VMEM((BT if const_ones else HB, G), jnp.int32),
            pltpu.VMEM((BT if const_ones else HB, G), jnp.int32),
            pltpu.VMEM((G, C), jnp.float32),
            pltpu.VMEM((G, C), jnp.float32),
            pltpu.VMEM_SHARED((NACC, C), jnp.float32),
            pltpu.SemaphoreType.DMA,
            pltpu.SemaphoreType.DMA,
            pltpu.SemaphoreType.DMA,
            pltpu.SemaphoreType.DMA,
        ],
    )
    def k(y_hbm, src_hbm, dst_hbm, zer_hbm, out_hbm, srcv, dstv, rows, rows2,
          zsp, sem, sem2, ssem, ssem2):
        c = lax.axis_index("c")
        s = lax.axis_index("s")
        base = (c * 16 + s) * BT
        if const_ones:
            pltpu.sync_copy(dst_hbm.at[pl.ds(base, BT)], dstv)

        def do_chunk(cc, oi):
            pltpu.sync_copy(zer_hbm, zsp.at[pl.ds(s * ZROWS, ZROWS)])
            if const_ones:
                pltpu.sync_copy(y_hbm.at[0], rows)
            plsc.subcore_barrier()

            if const_ones:
                def step(i, carry):
                    for u in range(U):
                        pltpu.sync_copy(rows, zsp.at[dstv.at[i * U + u]], add=True)
                    return carry

                lax.fori_loop(0, BT // U, step, 0)
            else:
                def gstart(j, buf, sm):
                    pltpu.async_copy(y_hbm.at[cc].at[srcv.at[j]], buf, sm)

                def gwait(buf, sm):
                    pltpu.make_async_copy(y_hbm.at[cc].at[srcv.at[0]], buf, sm).wait()

                def sstart(j, buf, sm):
                    pltpu.async_copy(buf, zsp.at[dstv.at[j]], sm, add=True)

                def swait(buf, sm):
                    pltpu.make_async_copy(buf, zsp.at[dstv.at[0]], sm).wait()

                for h in range(NH):
                    pltpu.sync_copy(src_hbm.at[pl.ds(base + h * HB, HB)], srcv)
                    pltpu.sync_copy(dst_hbm.at[pl.ds(base + h * HB, HB)], dstv)
                    gstart(0, rows, sem)
                    gstart(1, rows2, sem2)

                    def step(i, carry):
                        j = 2 * i
                        gwait(rows, sem)
                        sstart(j, rows, ssem)
                        gwait(rows2, sem2)
                        sstart(j + 1, rows2, ssem2)
                        swait(rows, ssem)

                        @pl.when(j + 2 < HB)
                        def _():
                            gstart(j + 2, rows, sem)

                        swait(rows2, ssem2)

                        @pl.when(j + 3 < HB)
                        def _():
                            gstart(j + 3, rows2, sem2)

                        return carry

                    lax.fori_loop(0, HB // 2, step, 0)
            plsc.subcore_barrier()
            pltpu.sync_copy(zsp.at[pl.ds(s * ZROWS, ZROWS)],
                            out_hbm.at[oi].at[pl.ds(s * ZROWS, ZROWS)])
            plsc.subcore_barrier()

        for i in range(nc):
            @pl.when(c == 0)
            def _():
                do_chunk(i, i)

            @pl.when(c == 1)
            def _():
                do_chunk(i, nc + i)

    return k


def _dinv_block(degz):
    d = degz[0, :, 0:1] + degz[1, :, 0:1] + 1.0
    return lax.rsqrt(d)


def _mm0(x, w0, degz):
    kb = x.shape[1]
    oc = w0.shape[1] // 128

    def body(deg_ref, x_ref, w_ref, o_ref):
        dinv = _dinv_block(deg_ref[...])
        o_ref[0] = jnp.dot(x_ref[...], w_ref[...],
                           preferred_element_type=jnp.float32) * dinv

    return pl.pallas_call(
        body,
        grid=(N // R, oc),
        in_specs=[
            pl.BlockSpec((2, R, 128), lambda r, o: (0, r, 0)),
            pl.BlockSpec((R, kb), lambda r, o: (r, 0)),
            pl.BlockSpec((kb, 128), lambda r, o: (0, o)),
        ],
        out_specs=pl.BlockSpec((1, R, 128), lambda r, o: (o, r, 0)),
        out_shape=jax.ShapeDtypeStruct((oc, N, 128), jnp.float32),
    )(degz, x, w0)


def _mmf(z, y, degz, b2d, w, co_chunk):
    kc = z.shape[0] // 2
    C = z.shape[2]
    oc = w.shape[1] // co_chunk

    def body(deg_ref, z_ref, zb_ref, y_ref, b_ref, w_ref, o_ref):
        k = pl.program_id(2)
        dinv = _dinv_block(deg_ref[...])
        t = jnp.maximum((z_ref[0] + zb_ref[0] + y_ref[0]) * dinv + b_ref[0], 0.0)
        part = jnp.dot(t, w_ref[...], preferred_element_type=jnp.float32)

        @pl.when(k == 0)
        def _():
            o_ref[0] = part

        @pl.when(k != 0)
        def _():
            o_ref[0] += part

        @pl.when(k == kc - 1)
        def _():
            o_ref[0] *= dinv

    return pl.pallas_call(
        body,
        grid=(N // R, oc, kc),
        in_specs=[
            pl.BlockSpec((2, R, 128), lambda r, o, k: (0, r, 0)),
            pl.BlockSpec((1, R, C), lambda r, o, k: (k, r, 0)),
            pl.BlockSpec((1, R, C), lambda r, o, k: (kc + k, r, 0)),
            pl.BlockSpec((1, R, C), lambda r, o, k: (k, r, 0)),
            pl.BlockSpec((1, 1, C), lambda r, o, k: (k, 0, 0)),
            pl.BlockSpec((C, co_chunk), lambda r, o, k: (k, o)),
        ],
        out_specs=pl.BlockSpec((1, R, co_chunk), lambda r, o, k: (o, r, 0)),
        out_shape=jax.ShapeDtypeStruct((oc, N, co_chunk), jnp.float32),
    )(degz, z, z, y, b2d, w)


def _epi(z2, y2, degz, b2):
    def body(deg_ref, z_ref, y_ref, b_ref, o_ref):
        dinv = _dinv_block(deg_ref[...])
        acc = z_ref[0, :, :64] + z_ref[1, :, :64] + y_ref[0, :, :64]
        o_ref[...] = acc * dinv + b_ref[...]

    return pl.pallas_call(
        body,
        grid=(N // R,),
        in_specs=[
            pl.BlockSpec((2, R, 128), lambda r: (0, r, 0)),
            pl.BlockSpec((2, R, 128), lambda r: (0, r, 0)),
            pl.BlockSpec((1, R, 128), lambda r: (0, r, 0)),
            pl.BlockSpec((1, 64), lambda r: (0, 0)),
        ],
        out_specs=pl.BlockSpec((R, 64), lambda r: (r, 0)),
        out_shape=jax.ShapeDtypeStruct((N, 64), jnp.float32),
    )(degz, z2, y2, b2)


def kernel(adj, features, W0, b0, W1, b1, W2, b2):
    pad = EPAD - E
    srcp = jnp.concatenate([adj[0], jnp.zeros((pad,), jnp.int32)]).reshape(NBLK, G)
    dstp = jnp.concatenate([adj[1], jnp.full((pad,), N, jnp.int32)]).reshape(NBLK, G)
    onesg = jnp.ones((1, G, 128), jnp.float32)
    z128 = jnp.zeros((ZROWS, 128), jnp.float32)
    W2p = jnp.pad(W2, ((0, 0), (0, 64)))

    degz = _sc_scatter(1, 128, True)(onesg, srcp, dstp, z128)
    y0 = _mm0(features, W0, degz)
    z0 = _sc_scatter(4, 128)(y0, srcp, dstp, z128)
    y1 = _mmf(z0, y0, degz, b0.reshape(4, 1, 128), W1, 128)
    z1 = _sc_scatter(4, 128)(y1, srcp, dstp, z128)
    y2 = _mmf(z1, y1, degz, b1.reshape(4, 1, 128), W2p, 128)
    z2 = _sc_scatter(1, 128)(y2, srcp, dstp, z128)
    return _epi(z2, y2, degz, b2.reshape(1, 64))


# speedup vs baseline: 1.3705x; 1.3705x over previous
import functools

import jax
import jax.numpy as jnp
from jax import lax
from jax.experimental import pallas as pl
from jax.experimental.pallas import tpu as pltpu
from jax.experimental.pallas import tpu_sc as plsc

N = 10000
E = 160000
G = 128
EPAD = 163840
NBLK = EPAD // G
NACC = 10112
ZROWS = NACC // 16
R = 2000


def _sc_scatter(nc, C, const_ones=False):
    two_out = nc == 1
    n_out = 2 if two_out else nc
    BT = NBLK // 32 if two_out else NBLK // 16
    HB = BT if two_out else BT // 2
    NH = BT // HB
    U = 8
    mesh = plsc.VectorSubcoreMesh(core_axis_name="c", subcore_axis_name="s",
                                  num_cores=2, num_subcores=16)

    @functools.partial(
        pl.kernel,
        out_type=jax.ShapeDtypeStruct((n_out, NACC, C), jnp.float32),
        mesh=mesh,
        scratch_types=[
            pltpu.VMEM((BT if const_ones else HB, G), jnp.int32),
            pltpu.VMEM((BT if const_ones else HB, G), jnp.int32),
            pltpu.VMEM((G, C), jnp.float32),
            pltpu.VMEM((G, C), jnp.float32),
            pltpu.VMEM_SHARED((NACC, C), jnp.float32),
            pltpu.SemaphoreType.DMA,
            pltpu.SemaphoreType.DMA,
            pltpu.SemaphoreType.DMA,
            pltpu.SemaphoreType.DMA,
        ],
    )
    def k(y_hbm, src_hbm, dst_hbm, zer_hbm, out_hbm, srcv, dstv, rows, rows2,
          zsp, sem, sem2, ssem, ssem2):
        c = lax.axis_index("c")
        s = lax.axis_index("s")
        base = (c * 16 + s) * BT if two_out else s * BT
        if const_ones:
            pltpu.sync_copy(dst_hbm.at[pl.ds(base, BT)], dstv)

        def do_chunk(cc, oi):
            pltpu.sync_copy(zer_hbm, zsp.at[pl.ds(s * ZROWS, ZROWS)])
            if const_ones:
                pltpu.sync_copy(y_hbm.at[0], rows)
            plsc.subcore_barrier()

            if const_ones:
                def step(i, carry):
                    for u in range(U):
                        pltpu.sync_copy(rows, zsp.at[dstv.at[i * U + u]], add=True)
                    return carry

                lax.fori_loop(0, BT // U, step, 0)
            else:
                def gstart(j, buf, sm):
                    pltpu.async_copy(y_hbm.at[cc].at[srcv.at[j]], buf, sm)

                def gwait(buf, sm):
                    pltpu.make_async_copy(y_hbm.at[cc].at[srcv.at[0]], buf, sm).wait()

                def sstart(j, buf, sm):
                    pltpu.async_copy(buf, zsp.at[dstv.at[j]], sm, add=True)

                def swait(buf, sm):
                    pltpu.make_async_copy(buf, zsp.at[dstv.at[0]], sm).wait()

                for h in range(NH):
                    pltpu.sync_copy(src_hbm.at[pl.ds(base + h * HB, HB)], srcv)
                    pltpu.sync_copy(dst_hbm.at[pl.ds(base + h * HB, HB)], dstv)
                    gstart(0, rows, sem)
                    gstart(1, rows2, sem2)

                    def step(i, carry):
                        j = 2 * i
                        gwait(rows, sem)
                        sstart(j, rows, ssem)
                        gwait(rows2, sem2)
                        sstart(j + 1, rows2, ssem2)
                        swait(rows, ssem)

                        @pl.when(j + 2 < HB)
                        def _():
                            gstart(j + 2, rows, sem)

                        swait(rows2, ssem2)

                        @pl.when(j + 3 < HB)
                        def _():
                            gstart(j + 3, rows2, sem2)

                        return carry

                    lax.fori_loop(0, HB // 2, step, 0)
            plsc.subcore_barrier()
            pltpu.sync_copy(zsp.at[pl.ds(s * ZROWS, ZROWS)],
                            out_hbm.at[oi].at[pl.ds(s * ZROWS, ZROWS)])
            plsc.subcore_barrier()

        if two_out:
            @pl.when(c == 0)
            def _():
                do_chunk(0, 0)

            @pl.when(c == 1)
            def _():
                do_chunk(0, 1)
        else:
            for i in range(nc // 2):
                @pl.when(c == 0)
                def _():
                    do_chunk(2 * i, 2 * i)

                @pl.when(c == 1)
                def _():
                    do_chunk(2 * i + 1, 2 * i + 1)

    return k


def _dinv_block(degz):
    d = degz[0, :, 0:1] + degz[1, :, 0:1] + 1.0
    return lax.rsqrt(d)


def _mm0(x, w0, degz):
    kb = x.shape[1]
    oc = w0.shape[1] // 128

    def body(deg_ref, x_ref, w_ref, o_ref):
        dinv = _dinv_block(deg_ref[...])
        o_ref[0] = jnp.dot(x_ref[...], w_ref[...],
                           preferred_element_type=jnp.float32) * dinv

    return pl.pallas_call(
        body,
        grid=(N // R, oc),
        in_specs=[
            pl.BlockSpec((2, R, 128), lambda r, o: (0, r, 0)),
            pl.BlockSpec((R, kb), lambda r, o: (r, 0)),
            pl.BlockSpec((kb, 128), lambda r, o: (0, o)),
        ],
        out_specs=pl.BlockSpec((1, R, 128), lambda r, o: (o, r, 0)),
        out_shape=jax.ShapeDtypeStruct((oc, N, 128), jnp.float32),
    )(degz, x, w0)


def _mmf(z, y, degz, b2d, w, co_chunk):
    kc, _, C = z.shape
    oc = w.shape[1] // co_chunk

    def body(deg_ref, z_ref, y_ref, b_ref, w_ref, o_ref):
        k = pl.program_id(2)
        dinv = _dinv_block(deg_ref[...])
        t = jnp.maximum((z_ref[0] + y_ref[0]) * dinv + b_ref[0], 0.0)
        part = jnp.dot(t, w_ref[...], preferred_element_type=jnp.float32)

        @pl.when(k == 0)
        def _():
            o_ref[0] = part

        @pl.when(k != 0)
        def _():
            o_ref[0] += part

        @pl.when(k == kc - 1)
        def _():
            o_ref[0] *= dinv

    return pl.pallas_call(
        body,
        grid=(N // R, oc, kc),
        in_specs=[
            pl.BlockSpec((2, R, 128), lambda r, o, k: (0, r, 0)),
            pl.BlockSpec((1, R, C), lambda r, o, k: (k, r, 0)),
            pl.BlockSpec((1, R, C), lambda r, o, k: (k, r, 0)),
            pl.BlockSpec((1, 1, C), lambda r, o, k: (k, 0, 0)),
            pl.BlockSpec((C, co_chunk), lambda r, o, k: (k, o)),
        ],
        out_specs=pl.BlockSpec((1, R, co_chunk), lambda r, o, k: (o, r, 0)),
        out_shape=jax.ShapeDtypeStruct((oc, N, co_chunk), jnp.float32),
    )(degz, z, y, b2d, w)


def _epi(z2, y2, degz, b2):
    def body(deg_ref, z_ref, y_ref, b_ref, o_ref):
        dinv = _dinv_block(deg_ref[...])
        acc = z_ref[0, :, :64] + z_ref[1, :, :64] + y_ref[0, :, :64]
        o_ref[...] = acc * dinv + b_ref[...]

    return pl.pallas_call(
        body,
        grid=(N // R,),
        in_specs=[
            pl.BlockSpec((2, R, 128), lambda r: (0, r, 0)),
            pl.BlockSpec((2, R, 128), lambda r: (0, r, 0)),
            pl.BlockSpec((1, R, 128), lambda r: (0, r, 0)),
            pl.BlockSpec((1, 64), lambda r: (0, 0)),
        ],
        out_specs=pl.BlockSpec((R, 64), lambda r: (r, 0)),
        out_shape=jax.ShapeDtypeStruct((N, 64), jnp.float32),
    )(degz, z2, y2, b2)


def kernel(adj, features, W0, b0, W1, b1, W2, b2):
    pad = EPAD - E
    srcp = jnp.concatenate([adj[0], jnp.zeros((pad,), jnp.int32)]).reshape(NBLK, G)
    dstp = jnp.concatenate([adj[1], jnp.full((pad,), N, jnp.int32)]).reshape(NBLK, G)
    onesg = jnp.ones((1, G, 128), jnp.float32)
    z128 = jnp.zeros((ZROWS, 128), jnp.float32)
    W2p = jnp.pad(W2, ((0, 0), (0, 64)))

    degz = _sc_scatter(1, 128, True)(onesg, srcp, dstp, z128)
    y0 = _mm0(features, W0, degz)
    z0 = _sc_scatter(4, 128)(y0, srcp, dstp, z128)
    y1 = _mmf(z0, y0, degz, b0.reshape(4, 1, 128), W1, 128)
    z1 = _sc_scatter(4, 128)(y1, srcp, dstp, z128)
    y2 = _mmf(z1, y1, degz, b1.reshape(4, 1, 128), W2p, 128)
    z2 = _sc_scatter(1, 128)(y2, srcp, dstp, z128)
    return _epi(z2, y2, degz, b2.reshape(1, 64))
